# baseline (device time: 49208 ns/iter reference)
import jax
import jax.numpy as jnp
from jax import lax
from jax.experimental import pallas as pl
from jax.experimental.pallas import tpu as pltpu

N_DEV = 4


def kernel(x, k, Wp):
    b, s, c = x.shape
    taps = k.shape[0]
    d_out = Wp.shape[1]
    half = s // 2

    f32 = jnp.float32
    bf16 = jnp.bfloat16

    def body(x_ref, k_ref, w_ref, out_ref, acc, st1, st2,
             send_sems, recv_sems):
        my = lax.axis_index("i")
        lo = my % 2
        hi = my // 2
        pA = my + 1 - 2 * lo
        pB = 3 - my

        u = ((my + 1) // 2) % 2
        kb0 = u
        nb0 = 1 - u
        kb1 = 2 + hi
        nb1 = 3 - hi

        barrier_sem = pltpu.get_barrier_semaphore()
        for nbr in (pA, pB):
            pl.semaphore_signal(
                barrier_sem, inc=1,
                device_id=(nbr,), device_id_type=pl.DeviceIdType.MESH,
            )
        pl.semaphore_wait(barrier_sem, 2)

        def partial_batch(bidx):
            xb = x_ref[bidx]
            acc_ = xb * k_ref[taps - 1][None, :]
            for t in range(taps - 1):
                sh = taps - 1 - t
                shifted = jnp.concatenate(
                    [jnp.zeros((sh, c), f32), xb[: s - sh, :]], axis=0
                )
                acc_ = acc_ + shifted * k_ref[t][None, :]
            a = acc_ * jax.nn.sigmoid(acc_)
            return jnp.dot(a, w_ref[...], preferred_element_type=f32)

        def send(src, dst, dev, sem):
            r = pltpu.make_async_remote_copy(
                src_ref=src, dst_ref=dst,
                send_sem=send_sems.at[sem], recv_sem=recv_sems.at[sem],
                device_id=(dev,), device_id_type=pl.DeviceIdType.MESH,
            )
            r.start()
            return r

        def recv(dst, dev, sem):
            return pltpu.make_async_remote_copy(
                src_ref=dst, dst_ref=dst,
                send_sem=send_sems.at[sem], recv_sem=recv_sems.at[sem],
                device_id=(dev,), device_id_type=pl.DeviceIdType.MESH,
            )

        c0 = pl.ds(0, half)
        c1 = pl.ds(half, half)

        out_ref[nb0] = partial_batch(nb0).astype(bf16)
        s1a0 = send(out_ref.at[nb0, c0], st1.at[0, c0], pA, 0)
        s1a1 = send(out_ref.at[nb0, c1], st1.at[0, c1], pA, 1)
        out_ref[nb1] = partial_batch(nb1).astype(bf16)
        s1b0 = send(out_ref.at[nb1, c0], st1.at[1, c0], pB, 2)
        s1b1 = send(out_ref.at[nb1, c1], st1.at[1, c1], pB, 3)

        acc[0] = partial_batch(kb0)
        acc[1] = partial_batch(kb1)

        s1a0.wait()
        acc[0, c0] = acc[0, c0] + st1[0, c0].astype(f32)
        out_ref[kb0, c0] = acc[0, c0].astype(bf16)
        s2a0 = send(out_ref.at[kb0, c0], st2.at[0, c0], pB, 4)
        s1b0.wait()
        acc[1, c0] = acc[1, c0] + st1[1, c0].astype(f32)
        out_ref[kb1, c0] = acc[1, c0].astype(bf16)
        s2b0 = send(out_ref.at[kb1, c0], st2.at[1, c0], pA, 6)
        s1a1.wait()
        acc[0, c1] = acc[0, c1] + st1[0, c1].astype(f32)
        out_ref[kb0, c1] = acc[0, c1].astype(bf16)
        s2a1 = send(out_ref.at[kb0, c1], st2.at[0, c1], pB, 5)
        s1b1.wait()
        acc[1, c1] = acc[1, c1] + st1[1, c1].astype(f32)
        out_ref[kb1, c1] = acc[1, c1].astype(bf16)
        s2b1 = send(out_ref.at[kb1, c1], st2.at[1, c1], pA, 7)

        s2a0.wait()
        acc[0, c0] = acc[0, c0] + st2[0, c0].astype(f32)
        out_ref[kb0, c0] = acc[0, c0].astype(bf16)
        s4a0 = send(out_ref.at[kb0, c0], out_ref.at[kb0, c0], pA, 8)
        s2b0.wait()
        acc[1, c0] = acc[1, c0] + st2[1, c0].astype(f32)
        out_ref[kb1, c0] = acc[1, c0].astype(bf16)
        s4b0 = send(out_ref.at[kb1, c0], out_ref.at[kb1, c0], pB, 10)
        s2a1.wait()
        acc[0, c1] = acc[0, c1] + st2[0, c1].astype(f32)
        out_ref[kb0, c1] = acc[0, c1].astype(bf16)
        s4a1 = send(out_ref.at[kb0, c1], out_ref.at[kb0, c1], pA, 9)
        s2b1.wait()
        acc[1, c1] = acc[1, c1] + st2[1, c1].astype(f32)
        out_ref[kb1, c1] = acc[1, c1].astype(bf16)
        s4b1 = send(out_ref.at[kb1, c1], out_ref.at[kb1, c1], pB, 11)

        r4a0 = recv(out_ref.at[nb0, c0], pA, 8)
        r4a1 = recv(out_ref.at[nb0, c1], pA, 9)
        r4b0 = recv(out_ref.at[nb1, c0], pB, 10)
        r4b1 = recv(out_ref.at[nb1, c1], pB, 11)
        for d in (s4a0, s4a1, s4b0, s4b1):
            d.wait_send()
        for d in (r4a0, r4a1, r4b0, r4b1):
            d.wait_recv()

    return pl.pallas_call(
        body,
        out_shape=jax.ShapeDtypeStruct((b, s, d_out), bf16),
        in_specs=[
            pl.BlockSpec(memory_space=pltpu.VMEM),
            pl.BlockSpec(memory_space=pltpu.VMEM),
            pl.BlockSpec(memory_space=pltpu.VMEM),
        ],
        out_specs=pl.BlockSpec(memory_space=pltpu.VMEM),
        scratch_shapes=[
            pltpu.VMEM((2, s, d_out), f32),
            pltpu.VMEM((2, s, d_out), bf16),
            pltpu.VMEM((2, s, d_out), bf16),
            pltpu.SemaphoreType.DMA((12,)),
            pltpu.SemaphoreType.DMA((12,)),
        ],
        compiler_params=pltpu.CompilerParams(collective_id=0),
    )(x, k, Wp)


# device time: 49058 ns/iter; 1.0031x vs baseline; 1.0031x over previous
import jax
import jax.numpy as jnp
from jax import lax
from jax.experimental import pallas as pl
from jax.experimental.pallas import tpu as pltpu

N_DEV = 4
NC = 4


def kernel(x, k, Wp):
    b, s, c = x.shape
    taps = k.shape[0]
    d_out = Wp.shape[1]
    cs = s // NC
    half = s // 2

    f32 = jnp.float32
    bf16 = jnp.bfloat16

    def body(x_ref, k_ref, w_ref, out_ref, acc, st1, st2,
             send_sems, recv_sems):
        my = lax.axis_index("i")
        lo = my % 2
        hi = my // 2
        pA = my + 1 - 2 * lo
        pB = 3 - my

        u = ((my + 1) // 2) % 2
        kb0 = u
        nb0 = 1 - u
        kb1 = 2 + hi
        nb1 = 3 - hi

        barrier_sem = pltpu.get_barrier_semaphore()
        for nbr in (pA, pB):
            pl.semaphore_signal(
                barrier_sem, inc=1,
                device_id=(nbr,), device_id_type=pl.DeviceIdType.MESH,
            )
        pl.semaphore_wait(barrier_sem, 2)

        def conv_silu(bidx):
            xb = x_ref[bidx]
            acc_ = xb * k_ref[taps - 1][None, :]
            for t in range(taps - 1):
                sh = taps - 1 - t
                shifted = jnp.concatenate(
                    [jnp.zeros((sh, c), f32), xb[: s - sh, :]], axis=0
                )
                acc_ = acc_ + shifted * k_ref[t][None, :]
            return acc_ * (1.0 / (1.0 + jnp.exp(-acc_)))

        def send(src, dst, dev, sem):
            r = pltpu.make_async_remote_copy(
                src_ref=src, dst_ref=dst,
                send_sem=send_sems.at[sem], recv_sem=recv_sems.at[sem],
                device_id=(dev,), device_id_type=pl.DeviceIdType.MESH,
            )
            r.start()
            return r

        def recv(dst, dev, sem):
            return pltpu.make_async_remote_copy(
                src_ref=dst, dst_ref=dst,
                send_sem=send_sems.at[sem], recv_sem=recv_sems.at[sem],
                device_id=(dev,), device_id_type=pl.DeviceIdType.MESH,
            )

        ch = [pl.ds(i * cs, cs) for i in range(NC)]
        def sem(ph, h, i):
            return ph * 2 * NC + h * NC + i

        a_n0 = conv_silu(nb0)
        s1a = []
        s1b = []
        for r0, lim in ((0, NC // 2), (half, NC)):
            out_ref[nb0, pl.ds(r0, half)] = jnp.dot(
                a_n0[r0:r0 + half, :], w_ref[...],
                preferred_element_type=f32,
            ).astype(bf16)
            while len(s1a) < lim:
                i = len(s1a)
                s1a.append(send(out_ref.at[nb0, ch[i]], st1.at[0, ch[i]],
                                pA, sem(0, 0, i)))
        a_n1 = conv_silu(nb1)
        for r0, lim in ((0, NC // 2), (half, NC)):
            out_ref[nb1, pl.ds(r0, half)] = jnp.dot(
                a_n1[r0:r0 + half, :], w_ref[...],
                preferred_element_type=f32,
            ).astype(bf16)
            while len(s1b) < lim:
                i = len(s1b)
                s1b.append(send(out_ref.at[nb1, ch[i]], st1.at[1, ch[i]],
                                pB, sem(0, 1, i)))

        acc[0] = jnp.dot(conv_silu(kb0), w_ref[...], preferred_element_type=f32)
        acc[1] = jnp.dot(conv_silu(kb1), w_ref[...], preferred_element_type=f32)

        s2a = []
        s2b = []
        for i in range(NC):
            s1a[i].wait()
            out_ref[kb0, ch[i]] = (
                acc[0, ch[i]] + st1[0, ch[i]].astype(f32)
            ).astype(bf16)
            s2a.append(send(out_ref.at[kb0, ch[i]], st2.at[0, ch[i]],
                            pB, sem(1, 0, i)))
            s1b[i].wait()
            out_ref[kb1, ch[i]] = (
                acc[1, ch[i]] + st1[1, ch[i]].astype(f32)
            ).astype(bf16)
            s2b.append(send(out_ref.at[kb1, ch[i]], st2.at[1, ch[i]],
                            pA, sem(1, 1, i)))

        s4a = []
        s4b = []
        for i in range(NC):
            s2a[i].wait()
            out_ref[kb0, ch[i]] = (
                acc[0, ch[i]] + st1[0, ch[i]].astype(f32)
                + st2[0, ch[i]].astype(f32)
            ).astype(bf16)
            s4a.append(send(out_ref.at[kb0, ch[i]], out_ref.at[kb0, ch[i]],
                            pA, sem(2, 0, i)))
            s2b[i].wait()
            out_ref[kb1, ch[i]] = (
                acc[1, ch[i]] + st1[1, ch[i]].astype(f32)
                + st2[1, ch[i]].astype(f32)
            ).astype(bf16)
            s4b.append(send(out_ref.at[kb1, ch[i]], out_ref.at[kb1, ch[i]],
                            pB, sem(2, 1, i)))

        r4a = [recv(out_ref.at[nb0, ch[i]], pA, sem(2, 0, i)) for i in range(NC)]
        r4b = [recv(out_ref.at[nb1, ch[i]], pB, sem(2, 1, i)) for i in range(NC)]
        for d in s4a + s4b:
            d.wait_send()
        for d in r4a + r4b:
            d.wait_recv()

    n_sems = 3 * 2 * NC
    return pl.pallas_call(
        body,
        out_shape=jax.ShapeDtypeStruct((b, s, d_out), bf16),
        in_specs=[
            pl.BlockSpec(memory_space=pltpu.VMEM),
            pl.BlockSpec(memory_space=pltpu.VMEM),
            pl.BlockSpec(memory_space=pltpu.VMEM),
        ],
        out_specs=pl.BlockSpec(memory_space=pltpu.VMEM),
        scratch_shapes=[
            pltpu.VMEM((2, s, d_out), f32),
            pltpu.VMEM((2, s, d_out), bf16),
            pltpu.VMEM((2, s, d_out), bf16),
            pltpu.SemaphoreType.DMA((n_sems,)),
            pltpu.SemaphoreType.DMA((n_sems,)),
        ],
        compiler_params=pltpu.CompilerParams(collective_id=0),
    )(x, k, Wp)


# device time: 48171 ns/iter; 1.0215x vs baseline; 1.0184x over previous
import jax
import jax.numpy as jnp
from jax import lax
from jax.experimental import pallas as pl
from jax.experimental.pallas import tpu as pltpu

N_DEV = 4
NC = 4


def kernel(x, k, Wp):
    b, s, c = x.shape
    taps = k.shape[0]
    d_out = Wp.shape[1]
    cs = s // NC
    half = s // 2

    f32 = jnp.float32
    bf16 = jnp.bfloat16

    def body(x_ref, k_ref, w_ref, out_ref, acc, st1, st2,
             send_sems, recv_sems):
        my = lax.axis_index("i")
        lo = my % 2
        hi = my // 2
        pA = my + 1 - 2 * lo
        pB = 3 - my

        u = ((my + 1) // 2) % 2
        kb0 = u
        nb0 = 1 - u
        kb1 = 2 + hi
        nb1 = 3 - hi

        barrier_sem = pltpu.get_barrier_semaphore()
        for nbr in (pA, pB):
            pl.semaphore_signal(
                barrier_sem, inc=1,
                device_id=(nbr,), device_id_type=pl.DeviceIdType.MESH,
            )
        pl.semaphore_wait(barrier_sem, 2)

        def conv_silu(bidx):
            xb = x_ref[bidx]
            acc_ = xb * k_ref[taps - 1][None, :]
            for t in range(taps - 1):
                sh = taps - 1 - t
                shifted = jnp.concatenate(
                    [jnp.zeros((sh, c), f32), xb[: s - sh, :]], axis=0
                )
                acc_ = acc_ + shifted * k_ref[t][None, :]
            return acc_ * (1.0 / (1.0 + jnp.exp(-acc_)))

        def send(src, dst, dev, sem):
            r = pltpu.make_async_remote_copy(
                src_ref=src, dst_ref=dst,
                send_sem=send_sems.at[sem], recv_sem=recv_sems.at[sem],
                device_id=(dev,), device_id_type=pl.DeviceIdType.MESH,
            )
            r.start()
            return r

        def recv(dst, dev, sem):
            return pltpu.make_async_remote_copy(
                src_ref=dst, dst_ref=dst,
                send_sem=send_sems.at[sem], recv_sem=recv_sems.at[sem],
                device_id=(dev,), device_id_type=pl.DeviceIdType.MESH,
            )

        ch = [pl.ds(i * cs, cs) for i in range(NC)]
        def sem(ph, h, i):
            return ph * 2 * NC + h * NC + i

        s1a = []
        s1b = []

        def mm_half(av, bidx, r0):
            out_ref[bidx, pl.ds(r0, half)] = jnp.dot(
                av[r0:r0 + half, :], w_ref[...],
                preferred_element_type=f32,
            ).astype(bf16)

        def issue_s1(descs, bidx, stslot, dev, h, lim):
            while len(descs) < lim:
                i = len(descs)
                descs.append(send(out_ref.at[bidx, ch[i]],
                                  st1.at[stslot, ch[i]], dev, sem(0, h, i)))

        a_n0 = conv_silu(nb0)
        mm_half(a_n0, nb0, 0)
        issue_s1(s1a, nb0, 0, pA, 0, NC // 2)
        a_n1 = conv_silu(nb1)
        mm_half(a_n1, nb1, 0)
        issue_s1(s1b, nb1, 1, pB, 1, NC // 2)
        mm_half(a_n0, nb0, half)
        issue_s1(s1a, nb0, 0, pA, 0, NC)
        mm_half(a_n1, nb1, half)
        issue_s1(s1b, nb1, 1, pB, 1, NC)

        acc[0] = jnp.dot(conv_silu(kb0), w_ref[...], preferred_element_type=f32)
        acc[1] = jnp.dot(conv_silu(kb1), w_ref[...], preferred_element_type=f32)

        s2a = []
        s2b = []
        for i in range(NC):
            s1a[i].wait()
            out_ref[kb0, ch[i]] = (
                acc[0, ch[i]] + st1[0, ch[i]].astype(f32)
            ).astype(bf16)
            s2a.append(send(out_ref.at[kb0, ch[i]], st2.at[0, ch[i]],
                            pB, sem(1, 0, i)))
            s1b[i].wait()
            out_ref[kb1, ch[i]] = (
                acc[1, ch[i]] + st1[1, ch[i]].astype(f32)
            ).astype(bf16)
            s2b.append(send(out_ref.at[kb1, ch[i]], st2.at[1, ch[i]],
                            pA, sem(1, 1, i)))

        s4a = []
        s4b = []
        for i in range(NC):
            s2a[i].wait()
            out_ref[kb0, ch[i]] = (
                acc[0, ch[i]] + st1[0, ch[i]].astype(f32)
                + st2[0, ch[i]].astype(f32)
            ).astype(bf16)
            s4a.append(send(out_ref.at[kb0, ch[i]], out_ref.at[kb0, ch[i]],
                            pA, sem(2, 0, i)))
            s2b[i].wait()
            out_ref[kb1, ch[i]] = (
                acc[1, ch[i]] + st1[1, ch[i]].astype(f32)
                + st2[1, ch[i]].astype(f32)
            ).astype(bf16)
            s4b.append(send(out_ref.at[kb1, ch[i]], out_ref.at[kb1, ch[i]],
                            pB, sem(2, 1, i)))

        r4a = [recv(out_ref.at[nb0, ch[i]], pA, sem(2, 0, i)) for i in range(NC)]
        r4b = [recv(out_ref.at[nb1, ch[i]], pB, sem(2, 1, i)) for i in range(NC)]
        for d in s4a + s4b:
            d.wait_send()
        for d in r4a + r4b:
            d.wait_recv()

    n_sems = 3 * 2 * NC
    return pl.pallas_call(
        body,
        out_shape=jax.ShapeDtypeStruct((b, s, d_out), bf16),
        in_specs=[
            pl.BlockSpec(memory_space=pltpu.VMEM),
            pl.BlockSpec(memory_space=pltpu.VMEM),
            pl.BlockSpec(memory_space=pltpu.VMEM),
        ],
        out_specs=pl.BlockSpec(memory_space=pltpu.VMEM),
        scratch_shapes=[
            pltpu.VMEM((2, s, d_out), f32),
            pltpu.VMEM((2, s, d_out), bf16),
            pltpu.VMEM((2, s, d_out), bf16),
            pltpu.SemaphoreType.DMA((n_sems,)),
            pltpu.SemaphoreType.DMA((n_sems,)),
        ],
        compiler_params=pltpu.CompilerParams(collective_id=0),
    )(x, k, Wp)
